# baseline (speedup 1.0000x reference)
import functools

import jax
import jax.numpy as jnp
from jax import lax
from jax.experimental import pallas as pl
from jax.experimental.pallas import tpu as pltpu
from jax.experimental.pallas import tpu_sc as plsc

_LANES = 128


def _build_sc_gather(B):
    info = plsc.get_sparse_core_info()
    NC, NS = info.num_cores, info.num_subcores
    NW = NC * NS
    bpw = B // NW
    nch = bpw // _LANES
    f32 = jnp.float32
    mesh = plsc.VectorSubcoreMesh(core_axis_name="c", subcore_axis_name="s")

    @functools.partial(
        pl.kernel,
        out_type=tuple(jax.ShapeDtypeStruct((B, _LANES), f32) for _ in range(6)),
        mesh=mesh,
        scratch_types=[pltpu.VMEM((nch, _LANES), jnp.int32) for _ in range(6)]
        + [pltpu.VMEM((_LANES, _LANES), f32) for _ in range(6)]
        + [pltpu.SemaphoreType.DMA, pltpu.SemaphoreType.DMA],
    )
    def sc_gather(s0, s1, s2, s3, s4, s5, t0, t1, t2, t3, t4, t5,
                  o0, o1, o2, o3, o4, o5,
                  v0, v1, v2, v3, v4, v5,
                  b0, b1, b2, b3, b4, b5, gsem, wsem):
        wid = lax.axis_index("s") * NC + lax.axis_index("c")
        base = wid * bpw
        sids = (s0, s1, s2, s3, s4, s5)
        tabs = (t0, t1, t2, t3, t4, t5)
        outs = (o0, o1, o2, o3, o4, o5)
        idxv = (v0, v1, v2, v3, v4, v5)
        bufs = (b0, b1, b2, b3, b4, b5)
        for s, v in zip(sids, idxv):
            pltpu.sync_copy(s.at[pl.ds(wid * nch, nch)], v)
        for j in range(nch):
            gets = [pltpu.async_copy(t.at[v.at[j]], b, gsem)
                    for t, v, b in zip(tabs, idxv, bufs)]
            for g in gets:
                g.wait()
            puts = [pltpu.async_copy(
                b, o.at[pl.ds(base + j * _LANES, _LANES)], wsem)
                for b, o in zip(bufs, outs)]
            for p in puts:
                p.wait()

    return sc_gather


def _repack_body(tin, tout):
    tout[...] = jnp.transpose(tin[...], (1, 0))


def _repack(tabT, V, D):
    CB = 1024
    grid = (V + CB - 1) // CB
    return pl.pallas_call(
        _repack_body,
        grid=(grid,),
        in_specs=[pl.BlockSpec((D, CB), lambda i: (0, i))],
        out_specs=pl.BlockSpec((CB, D), lambda i: (i, 0)),
        out_shape=jax.ShapeDtypeStruct((V, D), jnp.float32),
    )(tabT)


def _dense_body(uc, ic, g0, g1, g2, g3, g4, g5, au, ai, atu, ati, adu, adi,
                b6, wo, out):
    hp = jax.lax.Precision.HIGHEST
    dims = (16, 16, 64, 64, 32, 32)
    u = uc[...]
    i = ic[...]
    slabs = (g0[...], g1[...], g2[...], g3[...], g4[...], g5[...])
    mats = (au[...], ai[...], atu[...], ati[...], adu[...], adi[...])
    pre = b6[...]
    for k in range(6):
        d = dims[k]
        r = _LANES // d
        sel = lax.rem(u if k % 2 == 0 else i, r)
        feat = jnp.zeros_like(slabs[k][:, :d])
        for v in range(r):
            feat = jnp.where(sel == v, slabs[k][:, v * d:(v + 1) * d], feat)
        pre = pre + jnp.dot(feat, mats[k], precision=hp,
                            preferred_element_type=jnp.float32)
    x = jnp.maximum(pre, 0.0)
    out[...] = jnp.sum(x * wo[...], axis=1, keepdims=True)


def _dense(B, uc, ic, slabs, mats, b6, wo):
    BLK = 2048
    full = lambda shape: pl.BlockSpec(shape, lambda i: (0, 0))
    return pl.pallas_call(
        _dense_body,
        grid=(B // BLK,),
        in_specs=[pl.BlockSpec((BLK, 1), lambda i: (i, 0))] * 2
        + [pl.BlockSpec((BLK, _LANES), lambda i: (i, 0))] * 6
        + [full(m.shape) for m in mats] + [full((1, 8)), full((1, 8))],
        out_specs=pl.BlockSpec((BLK, 1), lambda i: (i, 0)),
        out_shape=jax.ShapeDtypeStruct((B, 1), jnp.float32),
    )(uc, ic, *slabs, *mats, b6, wo)


def kernel(user, item, user_genres, user_titles, user_dirs,
           item_genres, item_titles, item_dirs,
           W_g, b_g, W_t, b_t, W_d, b_d, W_out, b_out):
    B = user.shape[0]
    dims = (user_genres.shape[1], user_titles.shape[1], user_dirs.shape[1])
    user = user.astype(jnp.int32)
    item = item.astype(jnp.int32)

    tables = (user_genres, user_titles, user_dirs,
              item_genres, item_titles, item_dirs)
    tabs8, sids = [], []
    for k, t in enumerate(tables):
        d = dims[k % 3]
        r = _LANES // d
        rowmajor = _repack(t.T, t.shape[0], d)
        tabs8.append(rowmajor.reshape(-1, _LANES))
        idx = user if k < 3 else item
        sids.append((idx // r).reshape(B // _LANES, _LANES))

    slabs = _build_sc_gather(B)(*sids, *tabs8)

    mats = []
    for col, (W, d) in enumerate(((W_g, dims[0]), (W_t, dims[1]), (W_d, dims[2]))):
        for half in (0, 1):
            m = jnp.zeros((d, 8), jnp.float32)
            mats.append(m.at[:, 2 * col:2 * col + 2].set(W[:, half * d:(half + 1) * d].T))
    b6 = jnp.concatenate([b_g, b_t, b_d, jnp.ones((1,), jnp.float32),
                          jnp.zeros((1,), jnp.float32)]).reshape(1, 8)
    wo = jnp.concatenate([W_out[0], b_out, jnp.zeros((1,), jnp.float32)]).reshape(1, 8)

    ordered = (slabs[0], slabs[3], slabs[1], slabs[4], slabs[2], slabs[5])
    mats_o = (mats[0], mats[1], mats[2], mats[3], mats[4], mats[5])
    return _dense(B, user.reshape(B, 1), item.reshape(B, 1),
                  ordered, mats_o, b6, wo)


# speedup vs baseline: 4.4648x; 4.4648x over previous
import functools

import jax
import jax.numpy as jnp
from jax import lax
from jax.experimental import pallas as pl
from jax.experimental.pallas import tpu as pltpu
from jax.experimental.pallas import tpu_sc as plsc

_LANES = 128
_CB = 2048
_GRP = 8
_RR = _CB // _GRP


def _project_body(xg, xt, xd, wg, wt, wd, out):
    p = None
    for x, w in ((xg, wg), (xt, wt), (xd, wd)):
        d = jax.lax.dot_general(x[...], w[...], (((0,), (0,)), ((), ())),
                                preferred_element_type=jnp.float32)
        p = d if p is None else p + d
    out[...] = jnp.concatenate(
        [p[jj * _RR:(jj + 1) * _RR] for jj in range(_GRP)], axis=1)


def _project(tabTs, ws, V):
    grid = (V + _CB - 1) // _CB
    specs = [pl.BlockSpec((t.shape[0], _CB), lambda i: (0, i)) for t in tabTs]
    specs += [pl.BlockSpec(w.shape, lambda i: (0, 0)) for w in ws]
    return pl.pallas_call(
        _project_body,
        grid=(grid,),
        in_specs=specs,
        out_specs=pl.BlockSpec((_RR, _LANES), lambda i: (i, 0)),
        out_shape=jax.ShapeDtypeStruct((grid * _RR, _LANES), jnp.float32),
    )(*tabTs, *ws)


def _build_sc_gather(B):
    info = plsc.get_sparse_core_info()
    NC, NS = info.num_cores, info.num_subcores
    NW = NC * NS
    bpw = B // NW
    nch = bpw // _LANES
    f32 = jnp.float32
    mesh = plsc.VectorSubcoreMesh(core_axis_name="c", subcore_axis_name="s")

    @functools.partial(
        pl.kernel,
        out_type=tuple(jax.ShapeDtypeStruct((B, _LANES), f32) for _ in range(2)),
        mesh=mesh,
        scratch_types=[pltpu.VMEM((nch, _LANES), jnp.int32) for _ in range(2)]
        + [pltpu.VMEM((_LANES, _LANES), f32) for _ in range(2)]
        + [pltpu.SemaphoreType.DMA, pltpu.SemaphoreType.DMA],
    )
    def sc_gather(s0, s1, t0, t1, o0, o1, v0, v1, b0, b1, gsem, wsem):
        wid = lax.axis_index("s") * NC + lax.axis_index("c")
        base = wid * bpw
        for s, v in ((s0, v0), (s1, v1)):
            pltpu.sync_copy(s.at[pl.ds(wid * nch, nch)], v)
        for j in range(nch):
            gets = [pltpu.async_copy(t.at[v.at[j]], b, gsem)
                    for t, v, b in ((t0, v0, b0), (t1, v1, b1))]
            for g in gets:
                g.wait()
            puts = [pltpu.async_copy(
                b, o.at[pl.ds(base + j * _LANES, _LANES)], wsem)
                for b, o in ((b0, o0), (b1, o1))]
            for p in puts:
                p.wait()

    return sc_gather


def _dense_body(uc, ic, us, vs, b8, wo, out):
    u = uc[...]
    i = ic[...]
    uslab = us[...]
    islab = vs[...]
    pre = b8[...]
    for idx, slab in ((u, uslab), (i, islab)):
        sel = lax.rem(idx, _CB) // _RR
        ext = jnp.zeros_like(slab[:, :8])
        for v in range(_GRP):
            ext = jnp.where(sel == v, slab[:, v * 16:v * 16 + 8], ext)
        pre = pre + ext
    x = jnp.maximum(pre, 0.0)
    out[...] = jnp.sum(x * wo[...], axis=1, keepdims=True)


def _dense(B, uc, ic, uslabs, islabs, b8, wo):
    BLK = 2048
    full = lambda shape: pl.BlockSpec(shape, lambda i: (0, 0))
    return pl.pallas_call(
        _dense_body,
        grid=(B // BLK,),
        in_specs=[pl.BlockSpec((BLK, 1), lambda i: (i, 0))] * 2
        + [pl.BlockSpec((BLK, _LANES), lambda i: (i, 0))] * 2
        + [full((1, 8)), full((1, 8))],
        out_specs=pl.BlockSpec((BLK, 1), lambda i: (i, 0)),
        out_shape=jax.ShapeDtypeStruct((B, 1), jnp.float32),
    )(uc, ic, uslabs, islabs, b8, wo)


def kernel(user, item, user_genres, user_titles, user_dirs,
           item_genres, item_titles, item_dirs,
           W_g, b_g, W_t, b_t, W_d, b_d, W_out, b_out):
    B = user.shape[0]
    d_g, d_t, d_d = (user_genres.shape[1], user_titles.shape[1],
                     user_dirs.shape[1])
    user = user.astype(jnp.int32)
    item = item.astype(jnp.int32)

    def wpad(W, lo, hi, col):
        return jnp.zeros((hi - lo, 16), jnp.float32).at[:, 2 * col:2 * col + 2].set(
            W[:, lo:hi].T)
    wu = (wpad(W_g, 0, d_g, 0), wpad(W_t, 0, d_t, 1), wpad(W_d, 0, d_d, 2))
    wi = (wpad(W_g, d_g, 2 * d_g, 0), wpad(W_t, d_t, 2 * d_t, 1),
          wpad(W_d, d_d, 2 * d_d, 2))

    yu = _project((user_genres.T, user_titles.T, user_dirs.T), wu,
                  user_genres.shape[0])
    yi = _project((item_genres.T, item_titles.T, item_dirs.T), wi,
                  item_genres.shape[0])

    sids = []
    for idx in (user, item):
        sid = (idx // _CB) * _RR + lax.rem(idx, _CB) % _RR
        sids.append(sid.reshape(B // _LANES, _LANES))

    uslabs, islabs = _build_sc_gather(B)(sids[0], sids[1], yu, yi)

    b8 = jnp.concatenate([b_g, b_t, b_d, jnp.ones((1,), jnp.float32),
                          jnp.zeros((1,), jnp.float32)]).reshape(1, 8)
    wo = jnp.concatenate([W_out[0], b_out, jnp.zeros((1,), jnp.float32)]).reshape(1, 8)

    return _dense(B, user.reshape(B, 1), item.reshape(B, 1),
                  uslabs, islabs, b8, wo)


# speedup vs baseline: 7.2402x; 1.6216x over previous
import functools

import jax
import jax.numpy as jnp
from jax import lax
from jax.experimental import pallas as pl
from jax.experimental.pallas import tpu as pltpu
from jax.experimental.pallas import tpu_sc as plsc

_LANES = 128
_CB = 8192
_GRP = 8
_RR = _CB // _GRP


def _project_body(xg, xt, xd, wg, wt, wd, out):
    pT = None
    for x, w in ((xg, wg), (xt, wt), (xd, wd)):
        d = jax.lax.dot_general(w[...], x[...], (((0,), (0,)), ((), ())),
                                preferred_element_type=jnp.float32)
        pT = d if pT is None else pT + d
    out[...] = jnp.concatenate(
        [jnp.transpose(pT[:, jj * _RR:(jj + 1) * _RR], (1, 0))
         for jj in range(_GRP)], axis=1)


def _project(tabTs, ws, V):
    grid = (V + _CB - 1) // _CB
    specs = [pl.BlockSpec((t.shape[0], _CB), lambda i: (0, i)) for t in tabTs]
    specs += [pl.BlockSpec(w.shape, lambda i: (0, 0)) for w in ws]
    return pl.pallas_call(
        _project_body,
        grid=(grid,),
        in_specs=specs,
        out_specs=pl.BlockSpec((_RR, _LANES), lambda i: (i, 0)),
        out_shape=jax.ShapeDtypeStruct((grid * _RR, _LANES), jnp.float32),
    )(*tabTs, *ws)


def _build_sc_gather(B):
    info = plsc.get_sparse_core_info()
    NC, NS = info.num_cores, info.num_subcores
    NW = NC * NS
    bpw = B // NW
    nch = bpw // _LANES
    f32 = jnp.float32
    mesh = plsc.VectorSubcoreMesh(core_axis_name="c", subcore_axis_name="s")

    @functools.partial(
        pl.kernel,
        out_type=tuple(jax.ShapeDtypeStruct((B, _LANES), f32) for _ in range(2)),
        mesh=mesh,
        scratch_types=[pltpu.VMEM((nch, _LANES), jnp.int32) for _ in range(2)]
        + [pltpu.VMEM((_LANES, _LANES), f32) for _ in range(2)]
        + [pltpu.SemaphoreType.DMA, pltpu.SemaphoreType.DMA],
    )
    def sc_gather(s0, s1, t0, t1, o0, o1, v0, v1, b0, b1, gsem, wsem):
        wid = lax.axis_index("s") * NC + lax.axis_index("c")
        base = wid * bpw
        for s, v in ((s0, v0), (s1, v1)):
            pltpu.sync_copy(s.at[pl.ds(wid * nch, nch)], v)
        for j in range(nch):
            gets = [pltpu.async_copy(t.at[v.at[j]], b, gsem)
                    for t, v, b in ((t0, v0, b0), (t1, v1, b1))]
            for g in gets:
                g.wait()
            puts = [pltpu.async_copy(
                b, o.at[pl.ds(base + j * _LANES, _LANES)], wsem)
                for b, o in ((b0, o0), (b1, o1))]
            for p in puts:
                p.wait()

    return sc_gather


def _dense_body(uc, ic, us, vs, b8, wo, out):
    u = uc[...]
    i = ic[...]
    uslab = us[...]
    islab = vs[...]
    pre = b8[...]
    for idx, slab in ((u, uslab), (i, islab)):
        sel = lax.rem(idx, _CB) // _RR
        ext = jnp.zeros_like(slab[:, :8])
        for v in range(_GRP):
            ext = jnp.where(sel == v, slab[:, v * 16:v * 16 + 8], ext)
        pre = pre + ext
    x = jnp.maximum(pre, 0.0)
    out[...] = jnp.sum(x * wo[...], axis=1, keepdims=True)


def _dense(B, uc, ic, uslabs, islabs, b8, wo):
    BLK = 2048
    full = lambda shape: pl.BlockSpec(shape, lambda i: (0, 0))
    return pl.pallas_call(
        _dense_body,
        grid=(B // BLK,),
        in_specs=[pl.BlockSpec((BLK, 1), lambda i: (i, 0))] * 2
        + [pl.BlockSpec((BLK, _LANES), lambda i: (i, 0))] * 2
        + [full((1, 8)), full((1, 8))],
        out_specs=pl.BlockSpec((BLK, 1), lambda i: (i, 0)),
        out_shape=jax.ShapeDtypeStruct((B, 1), jnp.float32),
    )(uc, ic, uslabs, islabs, b8, wo)


def kernel(user, item, user_genres, user_titles, user_dirs,
           item_genres, item_titles, item_dirs,
           W_g, b_g, W_t, b_t, W_d, b_d, W_out, b_out):
    B = user.shape[0]
    d_g, d_t, d_d = (user_genres.shape[1], user_titles.shape[1],
                     user_dirs.shape[1])
    user = user.astype(jnp.int32)
    item = item.astype(jnp.int32)

    def wpad(W, lo, hi, col):
        return jnp.zeros((hi - lo, 16), jnp.float32).at[:, 2 * col:2 * col + 2].set(
            W[:, lo:hi].T)
    wu = (wpad(W_g, 0, d_g, 0), wpad(W_t, 0, d_t, 1), wpad(W_d, 0, d_d, 2))
    wi = (wpad(W_g, d_g, 2 * d_g, 0), wpad(W_t, d_t, 2 * d_t, 1),
          wpad(W_d, d_d, 2 * d_d, 2))

    yu = _project((user_genres.T, user_titles.T, user_dirs.T), wu,
                  user_genres.shape[0])
    yi = _project((item_genres.T, item_titles.T, item_dirs.T), wi,
                  item_genres.shape[0])

    sids = []
    for idx in (user, item):
        sid = (idx // _CB) * _RR + lax.rem(idx, _CB) % _RR
        sids.append(sid.reshape(B // _LANES, _LANES))

    uslabs, islabs = _build_sc_gather(B)(sids[0], sids[1], yu, yi)

    b8 = jnp.concatenate([b_g, b_t, b_d, jnp.ones((1,), jnp.float32),
                          jnp.zeros((1,), jnp.float32)]).reshape(1, 8)
    wo = jnp.concatenate([W_out[0], b_out, jnp.zeros((1,), jnp.float32)]).reshape(1, 8)

    return _dense(B, user.reshape(B, 1), item.reshape(B, 1),
                  uslabs, islabs, b8, wo)
